# baseline (speedup 1.0000x reference)
import functools

import jax
import jax.numpy as jnp
from jax import lax
from jax.experimental import pallas as pl
from jax.experimental.pallas import tpu as pltpu
from jax.experimental.pallas import tpu_sc as plsc

L = 16
NC, NS = 2, 16
NW = NC * NS
GB = 16
G = 4
NSG = GB // G

SC_FRAC = 0.08


def _make_sc_body(n, deg, d):
    kd = d // L
    row = deg * d
    gsz = G * row

    def _body(ft_hbm, a2_hbm, out_hbm, a2b, ftb, sv, ov):
        w = lax.axis_index("s") * NC + lax.axis_index("c")
        nblk = n // GB
        q, r = nblk // NW, nblk % NW
        b0 = w * q + jnp.minimum(w, r)
        nb = q + jnp.where(w < r, 1, 0)
        ng = nb * NSG
        g0 = b0 * NSG

        rows0 = lax.iota(jnp.int32, L)
        base_idx = rows0 * L

        def run(scope):
            sem_a, sem_f = scope

            def issue(g, slot):
                pltpu.async_copy(
                    a2_hbm.at[pl.ds(g * gsz, gsz)],
                    a2b.at[pl.ds(slot * gsz, gsz)], sem_a)
                pltpu.async_copy(
                    ft_hbm.at[pl.ds(g * gsz, gsz)],
                    ftb.at[pl.ds(slot * gsz, gsz)], sem_f)

            def wait(slot):
                pltpu.make_async_copy(
                    a2_hbm.at[pl.ds(0, gsz)],
                    a2b.at[pl.ds(slot * gsz, gsz)], sem_a).wait()
                pltpu.make_async_copy(
                    ft_hbm.at[pl.ds(0, gsz)],
                    ftb.at[pl.ds(slot * gsz, gsz)], sem_f).wait()

            issue(g0, 0)

            def blk_body(b, carry):
                def sg_body(sg, carry2):
                    g = b * NSG + sg
                    slot = lax.rem(g, 2)
                    wait(slot)
                    issue(g0 + jnp.minimum(g + 1, ng - 1), 1 - slot)

                    def node_body(j, carry3):
                        nbase = (slot * G + j) * row
                        for dg in range(deg):
                            cs = [a2b[pl.ds(nbase + (dg * d + k * L), L)]
                                  for k in range(kd)]
                            while len(cs) > 1:
                                cs = [cs[i] + cs[i + 1]
                                      for i in range(0, len(cs) - 1, 2)] + (
                                          [cs[-1]] if len(cs) % 2 else [])
                            sv[pl.ds(dg * L, L)] = cs[0]
                        s_halves = []
                        for half in range(deg // L):
                            off = half * L * L
                            gs = [plsc.load_gather(sv, [base_idx + (off + l)])
                                  for l in range(L)]
                            while len(gs) > 1:
                                gs = [gs[i] + gs[i + 1]
                                      for i in range(0, len(gs), 2)]
                            s_halves.append(gs[0])
                        e_halves = []
                        for s_ in s_halves:
                            e2 = jnp.exp(jnp.abs(s_) * 2.0)
                            t = 1.0 - 2.0 / (e2 + 1.0)
                            t = jnp.where(s_ < 0.0, -t, t)
                            e_halves.append(jnp.exp(t))
                        denom = e_halves[0]
                        for h in e_halves[1:]:
                            denom = denom + h
                        invv = 1.0 / lax.broadcast(jnp.sum(denom), (L,))
                        wn = [e * invv for e in e_halves]
                        accs = [jnp.zeros((L,), jnp.float32) for _ in range(kd)]
                        for dg in range(deg):
                            ws = wn[dg // L][dg % L]
                            for k in range(kd):
                                accs[k] = accs[k] + ws * ftb[
                                    pl.ds(nbase + (dg * d + k * L), L)]
                        obase = (sg * G + j) * d
                        for k in range(kd):
                            ov[pl.ds(obase + k * L, L)] = accs[k]
                        return carry3

                    lax.fori_loop(0, G, node_body, 0)
                    return carry2

                lax.fori_loop(0, NSG, sg_body, 0)
                pltpu.sync_copy(ov, out_hbm.at[pl.ds((b0 + b) * (GB * d), GB * d)])
                return carry

            lax.fori_loop(0, nb, blk_body, 0)
            wait(lax.rem(ng, 2))

        pl.run_scoped(
            run,
            [pltpu.SemaphoreType.DMA, pltpu.SemaphoreType.DMA],
        )

    return _body


def _sc_part(ft_flat, a2_flat, n_sc, deg, d):
    mesh = plsc.VectorSubcoreMesh(
        core_axis_name="c", subcore_axis_name="s", num_cores=NC, num_subcores=NS
    )
    run = pl.kernel(
        _make_sc_body(n_sc, deg, d),
        out_type=jax.ShapeDtypeStruct((n_sc * d,), jnp.float32),
        mesh=mesh,
        compiler_params=pltpu.CompilerParams(
            needs_layout_passes=False, skip_device_barrier=True),
        scratch_types=[
            pltpu.VMEM((2 * G * deg * d,), jnp.float32),
            pltpu.VMEM((2 * G * deg * d,), jnp.float32),
            pltpu.VMEM((deg * L,), jnp.float32),
            pltpu.VMEM((GB * d,), jnp.float32),
        ],
    )
    return run(ft_flat, a2_flat).reshape(n_sc, d)


def _tc_kernel_body(ft_ref, a2_ref, o_ref):
    a2s = jnp.sum(a2_ref[...], axis=-1, keepdims=True)
    e = jax.nn.softmax(jnp.tanh(a2s), axis=1)
    o_ref[...] = jnp.sum(e * ft_ref[...], axis=1)


def _tc_part(ft, a2, n_sc, bn):
    n, deg, d = ft.shape
    n_tc = n - n_sc
    blk0 = n_sc // bn
    grid = (n_tc // bn,)
    in_spec = pl.BlockSpec((bn, deg, d), lambda i: (blk0 + i, 0, 0))
    out_spec = pl.BlockSpec((bn, d), lambda i: (i, 0))
    f = pl.pallas_call(
        _tc_kernel_body,
        grid=grid,
        in_specs=[in_spec, in_spec],
        out_specs=out_spec,
        out_shape=jax.ShapeDtypeStruct((n_tc, d), jnp.float32),
    )
    return f(ft, a2)


def _pick_bn(n_sc, n_tc):
    for bn in (400, 320, 256, 240, 200, 160, 128, 120, 96, 80, 64, 48, 40, 32,
               24, 16, 8):
        if n_tc % bn == 0 and n_sc % bn == 0:
            return bn
    return 8


def kernel(ft, a2):
    n, deg, d = ft.shape
    n_sc = max(GB, int(n * SC_FRAC) // GB * GB)
    bn = _pick_bn(n_sc, n - n_sc)
    sc_out = _sc_part(ft.reshape(-1), a2.reshape(-1), n_sc, deg, d)
    tc_out = _tc_part(ft, a2, n_sc, bn)
    return jnp.concatenate([sc_out, tc_out], axis=0)


# speedup vs baseline: 1.0069x; 1.0069x over previous
import functools

import jax
import jax.numpy as jnp
from jax import lax
from jax.experimental import pallas as pl
from jax.experimental.pallas import tpu as pltpu
from jax.experimental.pallas import tpu_sc as plsc

L = 16
NC, NS = 2, 16
NW = NC * NS
GB = 16
G = 4
NSG = GB // G

SC_FRAC = 0.40


def _make_sc_body(n, deg, d):
    kd = d // L
    row = deg * d
    gsz = G * row

    def _body(ft_hbm, a2_hbm, out_hbm, a2b, ftb, sv, ov):
        w = lax.axis_index("s") * NC + lax.axis_index("c")
        nblk = n // GB
        q, r = nblk // NW, nblk % NW
        b0 = w * q + jnp.minimum(w, r)
        nb = q + jnp.where(w < r, 1, 0)
        ng = nb * NSG
        g0 = b0 * NSG

        rows0 = lax.iota(jnp.int32, L)
        base_idx = rows0 * L

        def run(scope):
            sem_a, sem_f = scope

            def issue(g, slot):
                pltpu.async_copy(
                    a2_hbm.at[pl.ds(g * gsz, gsz)],
                    a2b.at[pl.ds(slot * gsz, gsz)], sem_a)
                pltpu.async_copy(
                    ft_hbm.at[pl.ds(g * gsz, gsz)],
                    ftb.at[pl.ds(slot * gsz, gsz)], sem_f)

            def wait(slot):
                pltpu.make_async_copy(
                    a2_hbm.at[pl.ds(0, gsz)],
                    a2b.at[pl.ds(slot * gsz, gsz)], sem_a).wait()
                pltpu.make_async_copy(
                    ft_hbm.at[pl.ds(0, gsz)],
                    ftb.at[pl.ds(slot * gsz, gsz)], sem_f).wait()

            issue(g0, 0)

            def blk_body(b, carry):
                def sg_body(sg, carry2):
                    g = b * NSG + sg
                    slot = lax.rem(g, 2)
                    wait(slot)
                    issue(g0 + jnp.minimum(g + 1, ng - 1), 1 - slot)

                    def node_body(j, carry3):
                        nbase = (slot * G + j) * row
                        for dg in range(deg):
                            cs = [a2b[pl.ds(nbase + (dg * d + k * L), L)]
                                  for k in range(kd)]
                            while len(cs) > 1:
                                cs = [cs[i] + cs[i + 1]
                                      for i in range(0, len(cs) - 1, 2)] + (
                                          [cs[-1]] if len(cs) % 2 else [])
                            sv[pl.ds(dg * L, L)] = cs[0]
                        s_halves = []
                        for half in range(deg // L):
                            off = half * L * L
                            gs = [plsc.load_gather(sv, [base_idx + (off + l)])
                                  for l in range(L)]
                            while len(gs) > 1:
                                gs = [gs[i] + gs[i + 1]
                                      for i in range(0, len(gs), 2)]
                            s_halves.append(gs[0])
                        e_halves = []
                        for s_ in s_halves:
                            e2 = jnp.exp(jnp.abs(s_) * 2.0)
                            t = 1.0 - 2.0 / (e2 + 1.0)
                            t = jnp.where(s_ < 0.0, -t, t)
                            e_halves.append(jnp.exp(t))
                        denom = e_halves[0]
                        for h in e_halves[1:]:
                            denom = denom + h
                        invv = 1.0 / lax.broadcast(jnp.sum(denom), (L,))
                        wn = [e * invv for e in e_halves]
                        accs = [jnp.zeros((L,), jnp.float32) for _ in range(kd)]
                        for dg in range(deg):
                            ws = wn[dg // L][dg % L]
                            for k in range(kd):
                                accs[k] = accs[k] + ws * ftb[
                                    pl.ds(nbase + (dg * d + k * L), L)]
                        obase = (sg * G + j) * d
                        for k in range(kd):
                            ov[pl.ds(obase + k * L, L)] = accs[k]
                        return carry3

                    lax.fori_loop(0, G, node_body, 0)
                    return carry2

                lax.fori_loop(0, NSG, sg_body, 0)
                pltpu.sync_copy(ov, out_hbm.at[pl.ds((b0 + b) * (GB * d), GB * d)])
                return carry

            lax.fori_loop(0, nb, blk_body, 0)
            wait(lax.rem(ng, 2))

        pl.run_scoped(
            run,
            [pltpu.SemaphoreType.DMA, pltpu.SemaphoreType.DMA],
        )

    return _body


def _sc_part(ft_flat, a2_flat, n_sc, deg, d):
    mesh = plsc.VectorSubcoreMesh(
        core_axis_name="c", subcore_axis_name="s", num_cores=NC, num_subcores=NS
    )
    run = pl.kernel(
        _make_sc_body(n_sc, deg, d),
        out_type=jax.ShapeDtypeStruct((n_sc * d,), jnp.float32),
        mesh=mesh,
        compiler_params=pltpu.CompilerParams(
            needs_layout_passes=False, skip_device_barrier=True),
        scratch_types=[
            pltpu.VMEM((2 * G * deg * d,), jnp.float32),
            pltpu.VMEM((2 * G * deg * d,), jnp.float32),
            pltpu.VMEM((deg * L,), jnp.float32),
            pltpu.VMEM((GB * d,), jnp.float32),
        ],
    )
    return run(ft_flat, a2_flat).reshape(n_sc, d)


def _tc_kernel_body(ft_ref, a2_ref, o_ref):
    a2s = jnp.sum(a2_ref[...], axis=-1, keepdims=True)
    e = jax.nn.softmax(jnp.tanh(a2s), axis=1)
    o_ref[...] = jnp.sum(e * ft_ref[...], axis=1)


def _tc_part(ft, a2, n_sc, bn):
    n, deg, d = ft.shape
    n_tc = n - n_sc
    blk0 = n_sc // bn
    grid = (n_tc // bn,)
    in_spec = pl.BlockSpec((bn, deg, d), lambda i: (blk0 + i, 0, 0))
    out_spec = pl.BlockSpec((bn, d), lambda i: (i, 0))
    f = pl.pallas_call(
        _tc_kernel_body,
        grid=grid,
        in_specs=[in_spec, in_spec],
        out_specs=out_spec,
        out_shape=jax.ShapeDtypeStruct((n_tc, d), jnp.float32),
    )
    return f(ft, a2)


def _pick_bn(n_sc, n_tc):
    for bn in (400, 320, 256, 240, 200, 160, 128, 120, 96, 80, 64, 48, 40, 32,
               24, 16, 8):
        if n_tc % bn == 0 and n_sc % bn == 0:
            return bn
    return 8


def kernel(ft, a2):
    n, deg, d = ft.shape
    n_sc = max(GB, int(n * SC_FRAC) // GB * GB)
    bn = _pick_bn(n_sc, n - n_sc)
    tc_out = _tc_part(ft, a2, n_sc, bn)
    sc_out = _sc_part(ft.reshape(-1), a2.reshape(-1), n_sc, deg, d)
    return jnp.concatenate([sc_out, tc_out], axis=0)
